# Initial kernel scaffold; baseline (speedup 1.0000x reference)
import jax
import jax.numpy as jnp
from jax.experimental import pallas as pl
from jax.experimental.pallas import tpu as pltpu

_K = 16
_N = 4096
_C = 128
_R = 256


def _knn_tile_kernel(x_ref, out_ref):
    t = pl.program_id(1)
    xb = x_ref[0]

    norm = jnp.sqrt(jnp.sum(xb * xb, axis=1, keepdims=True))
    xn = xb / jnp.maximum(norm, 1e-12)

    xsq = jnp.sum(xn * xn, axis=0, keepdims=True)

    xq = jax.lax.dynamic_slice(xn, (0, t * _R), (_C, _R))
    xsq_q = jax.lax.dynamic_slice(xsq, (0, t * _R), (1, _R))

    inner = jax.lax.dot_general(
        xq, xn,
        dimension_numbers=(((0,), (0,)), ((), ())),
        preferred_element_type=jnp.float32,
        precision=jax.lax.Precision.HIGHEST,
    )

    d = (jnp.transpose(xsq_q) + (-2.0 * inner)) + xsq
    s = -d

    lane = jax.lax.broadcasted_iota(jnp.int32, (_R, _N), 1)
    for k in range(_K):
        idx = jnp.argmax(s, axis=1).astype(jnp.int32)
        out_ref[0, k, :] = idx
        s = jnp.where(lane == idx[:, None], -jnp.inf, s)


def _knn_indices(xb):
    grid = (xb.shape[0], _N // _R)
    return pl.pallas_call(
        _knn_tile_kernel,
        grid=grid,
        in_specs=[pl.BlockSpec((1, _C, _N), lambda b, t: (b, 0, 0))],
        out_specs=pl.BlockSpec((1, _K, _R), lambda b, t: (b, 0, t)),
        out_shape=jax.ShapeDtypeStruct((xb.shape[0], _K, _N), jnp.int32),
        compiler_params=pltpu.CompilerParams(
            dimension_semantics=("parallel", "arbitrary"),
        ),
    )(xb)


def kernel(x):
    b = x.shape[0]
    xb = jnp.squeeze(x, axis=-1)
    nn = jnp.transpose(_knn_indices(xb), (0, 2, 1))
    center = jnp.broadcast_to(
        jnp.arange(_N, dtype=jnp.int32)[None, :, None], (b, _N, _K)
    )
    return jnp.stack((nn, center), axis=0)


# speedup vs baseline: 13.3270x; 13.3270x over previous
import jax
import jax.numpy as jnp
from jax.experimental import pallas as pl
from jax.experimental.pallas import tpu as pltpu

_K = 16
_N = 4096
_C = 128
_R = 256


def _knn_tile_kernel(x_ref, out_ref):
    t = pl.program_id(1)
    xb = x_ref[0]

    norm = jnp.sqrt(jnp.sum(xb * xb, axis=1, keepdims=True))
    denom = jnp.maximum(norm, 1e-12)
    xn = xb / denom

    xsq = jnp.sum(xn * xn, axis=0, keepdims=True)

    xq = jnp.transpose(x_ref[0, :, pl.ds(t * _R, _R)] / denom)
    xsq_q = jnp.sum(xq * xq, axis=1, keepdims=True)

    inner = jax.lax.dot_general(
        xq.astype(jnp.bfloat16), xn.astype(jnp.bfloat16),
        dimension_numbers=(((1,), (0,)), ((), ())),
        preferred_element_type=jnp.float32,
    )

    d = (xsq_q + (-2.0 * inner)) + xsq
    s = -d

    lane = jax.lax.broadcasted_iota(jnp.int32, (_R, _N), 1)
    for k in range(_K):
        idx = jnp.argmax(s, axis=1).astype(jnp.int32)
        out_ref[0, k, :] = idx
        s = jnp.where(lane == idx[:, None], -jnp.inf, s)


def _knn_indices(xb):
    grid = (xb.shape[0], _N // _R)
    return pl.pallas_call(
        _knn_tile_kernel,
        grid=grid,
        in_specs=[pl.BlockSpec((1, _C, _N), lambda b, t: (b, 0, 0))],
        out_specs=pl.BlockSpec((1, _K, _R), lambda b, t: (b, 0, t)),
        out_shape=jax.ShapeDtypeStruct((xb.shape[0], _K, _N), jnp.int32),
        compiler_params=pltpu.CompilerParams(
            dimension_semantics=("parallel", "arbitrary"),
        ),
    )(xb)


def kernel(x):
    b = x.shape[0]
    xb = jnp.squeeze(x, axis=-1)
    nn = jnp.transpose(_knn_indices(xb), (0, 2, 1))
    center = jnp.broadcast_to(
        jnp.arange(_N, dtype=jnp.int32)[None, :, None], (b, _N, _K)
    )
    return jnp.stack((nn, center), axis=0)


# speedup vs baseline: 13.3323x; 1.0004x over previous
import jax
import jax.numpy as jnp
from jax.experimental import pallas as pl
from jax.experimental.pallas import tpu as pltpu

_K = 16
_N = 4096
_C = 128
_R = 256


def _knn_tile_kernel(x_ref, out_ref):
    t = pl.program_id(1)
    xb = x_ref[0]

    norm = jnp.sqrt(jnp.sum(xb * xb, axis=1, keepdims=True))
    denom = jnp.maximum(norm, 1e-12)
    xn = xb / denom

    xsq = jnp.sum(xn * xn, axis=0, keepdims=True)

    xq = jnp.transpose(x_ref[0, :, pl.ds(t * _R, _R)] / denom)
    xsq_q = jnp.sum(xq * xq, axis=1, keepdims=True)

    inner = jax.lax.dot_general(
        xq.astype(jnp.bfloat16), xn.astype(jnp.bfloat16),
        dimension_numbers=(((1,), (0,)), ((), ())),
        preferred_element_type=jnp.float32,
    )

    d = (xsq_q + (-2.0 * inner)) + xsq
    s = -d

    lane = jax.lax.broadcasted_iota(jnp.int32, (_R, _N), 1)
    for k in range(_K):
        idx = jnp.argmax(s, axis=1).astype(jnp.int32)
        out_ref[0, k, :] = idx
        s = jnp.where(lane == idx[:, None], -jnp.inf, s)


def _knn_indices(xb):
    grid = (xb.shape[0], _N // _R)
    return pl.pallas_call(
        _knn_tile_kernel,
        grid=grid,
        in_specs=[pl.BlockSpec((1, _C, _N), lambda b, t: (b, 0, 0))],
        out_specs=pl.BlockSpec((1, _K, _R), lambda b, t: (b, 0, t)),
        out_shape=jax.ShapeDtypeStruct((xb.shape[0], _K, _N), jnp.int32),
        compiler_params=pltpu.CompilerParams(
            dimension_semantics=("parallel", "parallel"),
        ),
    )(xb)


def kernel(x):
    b = x.shape[0]
    xb = jnp.squeeze(x, axis=-1)
    nn = jnp.transpose(_knn_indices(xb), (0, 2, 1))
    center = jnp.broadcast_to(
        jnp.arange(_N, dtype=jnp.int32)[None, :, None], (b, _N, _K)
    )
    return jnp.stack((nn, center), axis=0)
